# Initial kernel scaffold; baseline (speedup 1.0000x reference)
import jax
import jax.numpy as jnp
from jax.experimental import pallas as pl


def kernel(edge_index_view, edge_index_save, edge_index_buy, user_ids, item_ids, user_table, item_table, W1_ui, b1_ui, W1_iu, b1_iu, W2_ui, b2_ui, W2_iu, b2_iu, dec_w1, dec_b1, dec_w2, dec_b2):
    raise NotImplementedError("write your pallas kernel here")


# speedup vs baseline: 2.1376x; 2.1376x over previous
import functools

import jax
import jax.numpy as jnp
from jax import lax
from jax.experimental import pallas as pl
from jax.experimental.pallas import tpu as pltpu
from jax.experimental.pallas import tpu_sc as plsc

NU = 50000
NI = 50000
D = 128
E = 500000
B = 16384

NC, NS = 2, 16
NW = NC * NS
NP = 50048
RPT16 = NP // NS
RPT32 = NP // NW
PAD_IDX = 50000
K = 128
NBATCH = 246
EP = NS * NBATCH * K
CW = 8
NCHUNK = D // CW
CPS = NCHUNK // NC
ZROWS = RPT16 // 2

_mesh = plsc.VectorSubcoreMesh(core_axis_name="c", subcore_axis_name="s",
                               num_cores=NC, num_subcores=NS)
_sc_params = pltpu.CompilerParams(use_tc_tiling_on_sc=False)


@functools.partial(
    pl.kernel,
    out_type=jax.ShapeDtypeStruct((6, NP, 8), jnp.float32),
    mesh=_mesh,
    compiler_params=_sc_params,
    scratch_types=[
        pltpu.VMEM((NBATCH, K), jnp.int32),
        pltpu.VMEM((K, 8), jnp.float32),
        pltpu.VMEM((ZROWS, 8), jnp.float32),
        pltpu.VMEM_SHARED((NP, 8), jnp.float32),
        pltpu.SemaphoreType.DMA,
    ],
)
def _degree_kernel(ei_hbm, ones_hbm, zeros_hbm, deg_hbm,
                   idx_v, ones_v, zbuf_v, acc_sh, sem):
    cid = lax.axis_index("c")
    sid = lax.axis_index("s")
    wid = cid * NS + sid
    pltpu.sync_copy(ones_hbm, ones_v)
    pltpu.sync_copy(zeros_hbm, zbuf_v)
    for r in range(3):
        for e in range(2):
            for h in range(2):
                pltpu.sync_copy(
                    zbuf_v, acc_sh.at[pl.ds(sid * RPT16 + h * ZROWS, ZROWS)])
            plsc.subcore_barrier()
            pltpu.sync_copy(ei_hbm.at[r, e, sid], idx_v)

            def body(b):
                pltpu.sync_copy(ones_v, acc_sh.at[idx_v.at[b]], add=True)
            pl.loop(0, NBATCH)(body)
            plsc.subcore_barrier()
            sl = pl.ds(wid * RPT32, RPT32)
            pltpu.sync_copy(acc_sh.at[sl], deg_hbm.at[2 * r + e].at[sl])
            plsc.subcore_barrier()


@functools.partial(
    pl.kernel,
    out_type=jax.ShapeDtypeStruct((6, NCHUNK, NP, CW), jnp.float32),
    mesh=_mesh,
    compiler_params=_sc_params,
    scratch_types=[
        pltpu.VMEM((NBATCH, K), jnp.int32),
        pltpu.VMEM((NBATCH, K), jnp.int32),
        pltpu.VMEM((K, CW), jnp.float32),
        pltpu.VMEM((K, CW), jnp.float32),
        pltpu.VMEM((ZROWS, CW), jnp.float32),
        pltpu.VMEM_SHARED((NP, CW), jnp.float32),
        pltpu.SemaphoreType.DMA,
        pltpu.SemaphoreType.DMA,
    ],
)
def _propagate_kernel(y_u_hbm, y_i_hbm, ei_hbm, zeros_hbm, agg_hbm,
                      gidx_v, sidx_v, rows0_v, rows1_v, zbuf_v, acc_sh,
                      sem0, sem1):
    cid = lax.axis_index("c")
    sid = lax.axis_index("s")
    pltpu.sync_copy(zeros_hbm, zbuf_v)
    for p in range(6):
        r = p % 3
        gd, sd = (0, 1) if p < 3 else (1, 0)
        ytab = y_u_hbm if p < 3 else y_i_hbm
        pltpu.sync_copy(ei_hbm.at[r, gd, sid], gidx_v)
        pltpu.sync_copy(ei_hbm.at[r, sd, sid], sidx_v)
        for j in range(CPS):
            f = CPS * cid + j
            for z in range(2):
                pltpu.sync_copy(
                    zbuf_v, acc_sh.at[pl.ds(sid * RPT16 + z * ZROWS, ZROWS)])
            plsc.subcore_barrier()
            ytab_rf = ytab.at[r, f]
            pltpu.async_copy(ytab_rf.at[gidx_v.at[0]], rows0_v, sem0)

            def body(b):
                pltpu.async_copy(ytab_rf.at[gidx_v.at[b + 1]], rows1_v, sem1)
                pltpu.make_async_copy(ytab_rf.at[gidx_v.at[b]], rows0_v,
                                      sem0).wait()
                pltpu.sync_copy(rows0_v, acc_sh.at[sidx_v.at[b]], add=True)

                @pl.when(b + 2 < NBATCH)
                def _():
                    pltpu.async_copy(ytab_rf.at[gidx_v.at[b + 2]], rows0_v,
                                     sem0)
                pltpu.make_async_copy(ytab_rf.at[gidx_v.at[b + 1]], rows1_v,
                                      sem1).wait()
                pltpu.sync_copy(rows1_v, acc_sh.at[sidx_v.at[b + 1]], add=True)
            pl.loop(0, NBATCH, step=2)(body)
            plsc.subcore_barrier()
            sl = pl.ds(sid * RPT16, RPT16)
            pltpu.sync_copy(acc_sh.at[sl], agg_hbm.at[p, f].at[sl])
            plsc.subcore_barrier()


@functools.partial(
    pl.kernel,
    out_type=[jax.ShapeDtypeStruct((3, NCHUNK, B, CW), jnp.float32),
              jax.ShapeDtypeStruct((3, NCHUNK, B, CW), jnp.float32),
              jax.ShapeDtypeStruct((B, 8), jnp.float32),
              jax.ShapeDtypeStruct((B, 8), jnp.float32)],
    mesh=_mesh,
    compiler_params=_sc_params,
    scratch_types=[
        pltpu.VMEM((4, K), jnp.int32),
        pltpu.VMEM((4, K), jnp.int32),
        pltpu.VMEM((K, CW), jnp.float32),
        pltpu.VMEM((K, 8), jnp.float32),
        pltpu.SemaphoreType.DMA,
    ],
)
def _gather_kernel(agg_hbm, isdp_u_hbm, isdp_i_hbm, uid_hbm, iid_hbm,
                   gu_hbm, gi_hbm, gisd_u_hbm, gisd_i_hbm,
                   uidx_v, iidx_v, rows_v, rows8_v, sem):
    cid = lax.axis_index("c")
    sid = lax.axis_index("s")
    wid = cid * NS + sid
    pltpu.sync_copy(uid_hbm.at[wid], uidx_v)
    pltpu.sync_copy(iid_hbm.at[wid], iidx_v)
    for b in range(4):
        osl = pl.ds(wid * 4 * K + b * K, K)
        for r in range(3):
            for f in range(NCHUNK):
                pltpu.async_copy(agg_hbm.at[3 + r, f].at[uidx_v.at[b]],
                                 rows_v, sem).wait()
                pltpu.sync_copy(rows_v, gu_hbm.at[r, f].at[osl])
                pltpu.async_copy(agg_hbm.at[r, f].at[iidx_v.at[b]],
                                 rows_v, sem).wait()
                pltpu.sync_copy(rows_v, gi_hbm.at[r, f].at[osl])
        pltpu.async_copy(isdp_u_hbm.at[uidx_v.at[b]], rows8_v, sem).wait()
        pltpu.sync_copy(rows8_v, gisd_u_hbm.at[osl])
        pltpu.async_copy(isdp_i_hbm.at[iidx_v.at[b]], rows8_v, sem).wait()
        pltpu.sync_copy(rows8_v, gisd_i_hbm.at[osl])


def _prep_body(ut_ref, it_ref, deg_ref, yu_ref, yi_ref, pu_ref, pi_ref):
    ut = ut_ref[...]
    it = it_ref[...]
    deg = deg_ref[...]
    isd_u, isd_i = [], []
    for r in range(3):
        isd_u.append(lax.rsqrt(jnp.maximum(deg[2 * r, :, 0:1], 1.0)))
        isd_i.append(lax.rsqrt(jnp.maximum(deg[2 * r + 1, :, 0:1], 1.0)))
        for f in range(NCHUNK):
            yu_ref[r, f] = ut[:, CW * f:CW * f + CW] * isd_u[r]
            yi_ref[r, f] = it[:, CW * f:CW * f + CW] * isd_i[r]
    pad = jnp.ones((isd_u[0].shape[0], 5), jnp.float32)
    pu_ref[...] = jnp.concatenate(isd_u + [pad], axis=1)
    pi_ref[...] = jnp.concatenate(isd_i + [pad], axis=1)


_prep_tc = pl.pallas_call(
    _prep_body,
    grid=(NP // 128,),
    in_specs=[
        pl.BlockSpec((128, 128), lambda i: (i, 0)),
        pl.BlockSpec((128, 128), lambda i: (i, 0)),
        pl.BlockSpec((6, 128, 8), lambda i: (0, i, 0)),
    ],
    out_specs=[
        pl.BlockSpec((3, NCHUNK, 128, CW), lambda i: (0, 0, i, 0)),
        pl.BlockSpec((3, NCHUNK, 128, CW), lambda i: (0, 0, i, 0)),
        pl.BlockSpec((128, 8), lambda i: (i, 0)),
        pl.BlockSpec((128, 8), lambda i: (i, 0)),
    ],
    out_shape=[
        jax.ShapeDtypeStruct((3, NCHUNK, NP, CW), jnp.float32),
        jax.ShapeDtypeStruct((3, NCHUNK, NP, CW), jnp.float32),
        jax.ShapeDtypeStruct((NP, 8), jnp.float32),
        jax.ShapeDtypeStruct((NP, 8), jnp.float32),
    ],
)


def _post1_body(agg_ref, pu_ref, pi_ref, wui_ref, bui_ref, wiu_ref, biu_ref,
                y2u_ref, y2i_ref):
    wui = wui_ref[...]
    bui = bui_ref[...]
    wiu = wiu_ref[...]
    biu = biu_ref[...]
    pu = pu_ref[...]
    pi = pi_ref[...]
    i1 = None
    u1 = None
    for r in range(3):
        ai = jnp.concatenate([agg_ref[r, f] for f in range(NCHUNK)],
                             axis=-1) * pi[:, r:r + 1]
        au = jnp.concatenate([agg_ref[3 + r, f] for f in range(NCHUNK)],
                             axis=-1) * pu[:, r:r + 1]
        oi = jnp.dot(ai, wui[r], preferred_element_type=jnp.float32) \
            + bui[r:r + 1, :]
        ou = jnp.dot(au, wiu[r], preferred_element_type=jnp.float32) \
            + biu[r:r + 1, :]
        i1 = oi if i1 is None else jnp.maximum(i1, oi)
        u1 = ou if u1 is None else jnp.maximum(u1, ou)
    i1 = jnp.maximum(i1, 0.0)
    u1 = jnp.maximum(u1, 0.0)
    for r in range(3):
        for f in range(NCHUNK):
            y2u_ref[r, f] = u1[:, CW * f:CW * f + CW] * pu[:, r:r + 1]
            y2i_ref[r, f] = i1[:, CW * f:CW * f + CW] * pi[:, r:r + 1]


_post1_tc = pl.pallas_call(
    _post1_body,
    grid=(NP // 128,),
    in_specs=[
        pl.BlockSpec((6, NCHUNK, 128, CW), lambda i: (0, 0, i, 0)),
        pl.BlockSpec((128, 8), lambda i: (i, 0)),
        pl.BlockSpec((128, 8), lambda i: (i, 0)),
        pl.BlockSpec((3, 128, 128), lambda i: (0, 0, 0)),
        pl.BlockSpec((3, 128), lambda i: (0, 0)),
        pl.BlockSpec((3, 128, 128), lambda i: (0, 0, 0)),
        pl.BlockSpec((3, 128), lambda i: (0, 0)),
    ],
    out_specs=[
        pl.BlockSpec((3, NCHUNK, 128, CW), lambda i: (0, 0, i, 0)),
        pl.BlockSpec((3, NCHUNK, 128, CW), lambda i: (0, 0, i, 0)),
    ],
    out_shape=[
        jax.ShapeDtypeStruct((3, NCHUNK, NP, CW), jnp.float32),
        jax.ShapeDtypeStruct((3, NCHUNK, NP, CW), jnp.float32),
    ],
)


def _decoder_body(gu_ref, gi_ref, su_ref, si_ref, wui_ref, bui_ref,
                  wiu_ref, biu_ref, dw1_ref, db1_ref, dw2_ref, db2_ref,
                  out_ref):
    wui = wui_ref[...]
    wiu = wiu_ref[...]
    su = su_ref[...]
    si = si_ref[...]
    u2 = jnp.sum(biu_ref[...], axis=0, keepdims=True)
    i2 = jnp.sum(bui_ref[...], axis=0, keepdims=True)
    for r in range(3):
        au = jnp.concatenate([gu_ref[r, f] for f in range(NCHUNK)],
                             axis=-1) * su[:, r:r + 1]
        ai = jnp.concatenate([gi_ref[r, f] for f in range(NCHUNK)],
                             axis=-1) * si[:, r:r + 1]
        u2 = u2 + jnp.dot(au, wiu[r], preferred_element_type=jnp.float32)
        i2 = i2 + jnp.dot(ai, wui[r], preferred_element_type=jnp.float32)
    dw1 = dw1_ref[...]
    h = (jnp.dot(u2, dw1[:128, :], preferred_element_type=jnp.float32)
         + jnp.dot(i2, dw1[128:, :], preferred_element_type=jnp.float32)
         + db1_ref[...])
    h = jnp.maximum(h, 0.0)
    out_ref[...] = (jnp.dot(h, dw2_ref[...],
                            preferred_element_type=jnp.float32)
                    + db2_ref[...])


_decoder_tc = pl.pallas_call(
    _decoder_body,
    grid=(B // 128,),
    in_specs=[
        pl.BlockSpec((3, NCHUNK, 128, CW), lambda i: (0, 0, i, 0)),
        pl.BlockSpec((3, NCHUNK, 128, CW), lambda i: (0, 0, i, 0)),
        pl.BlockSpec((128, 8), lambda i: (i, 0)),
        pl.BlockSpec((128, 8), lambda i: (i, 0)),
        pl.BlockSpec((3, 128, 128), lambda i: (0, 0, 0)),
        pl.BlockSpec((3, 128), lambda i: (0, 0)),
        pl.BlockSpec((3, 128, 128), lambda i: (0, 0, 0)),
        pl.BlockSpec((3, 128), lambda i: (0, 0)),
        pl.BlockSpec((256, 128), lambda i: (0, 0)),
        pl.BlockSpec((1, 128), lambda i: (0, 0)),
        pl.BlockSpec((128, 4), lambda i: (0, 0)),
        pl.BlockSpec((1, 4), lambda i: (0, 0)),
    ],
    out_specs=pl.BlockSpec((128, 4), lambda i: (i, 0)),
    out_shape=jax.ShapeDtypeStruct((B, 4), jnp.float32),
)


def kernel(edge_index_view, edge_index_save, edge_index_buy, user_ids,
           item_ids, user_table, item_table,
           W1_ui, b1_ui, W1_iu, b1_iu, W2_ui, b2_ui, W2_iu, b2_iu,
           dec_w1, dec_b1, dec_w2, dec_b2):
    def prep_rel(ei):
        pad = jnp.full((EP - E,), PAD_IDX, jnp.int32)
        s = jnp.concatenate([ei[0].astype(jnp.int32), pad])
        d = jnp.concatenate([ei[1].astype(jnp.int32), pad])
        return jnp.stack([s.reshape(NS, NBATCH, K), d.reshape(NS, NBATCH, K)])

    ei = jnp.stack([prep_rel(edge_index_view), prep_rel(edge_index_save),
                    prep_rel(edge_index_buy)])
    ones8 = jnp.ones((K, 8), jnp.float32)
    zeros_cw = jnp.zeros((ZROWS, CW), jnp.float32)
    zeros8 = jnp.zeros((ZROWS, 8), jnp.float32)
    ut = jnp.pad(user_table, ((0, NP - NU), (0, 0)))
    it = jnp.pad(item_table, ((0, NP - NI), (0, 0)))
    uid = user_ids.astype(jnp.int32).reshape(NW, 4, K)
    iid = item_ids.astype(jnp.int32).reshape(NW, 4, K)

    deg = _degree_kernel(ei, ones8, zeros8)
    y1u, y1i, pu, pi = _prep_tc(ut, it, deg)
    agg1 = _propagate_kernel(y1u, y1i, ei, zeros_cw)
    y2u, y2i = _post1_tc(agg1, pu, pi, W1_ui, b1_ui, W1_iu, b1_iu)
    agg2 = _propagate_kernel(y2u, y2i, ei, zeros_cw)
    gu, gi, gsu, gsi = _gather_kernel(agg2, pu, pi, uid, iid)
    logits = _decoder_tc(gu, gi, gsu, gsi, W2_ui, b2_ui, W2_iu, b2_iu,
                         dec_w1, dec_b1.reshape(1, 128), dec_w2,
                         dec_b2.reshape(1, 4))
    return logits


# speedup vs baseline: 2.5001x; 1.1696x over previous
import functools

import jax
import jax.numpy as jnp
from jax import lax
from jax.experimental import pallas as pl
from jax.experimental.pallas import tpu as pltpu
from jax.experimental.pallas import tpu_sc as plsc

NU = 50000
NI = 50000
D = 128
E = 500000
B = 16384

NC, NS = 2, 16
NW = NC * NS
NP = 50048
RPT16 = NP // NS
RPT32 = NP // NW
PAD_IDX = 50000
K = 128
NBATCH = 246
EPT = NBATCH * K
EP = NS * EPT
CW = 8
NCHUNK = D // CW
CPS = NCHUNK // NC
ZROWS = RPT16 // 2

_mesh = plsc.VectorSubcoreMesh(core_axis_name="c", subcore_axis_name="s",
                               num_cores=NC, num_subcores=NS)
_sc_params = pltpu.CompilerParams(use_tc_tiling_on_sc=False)


@functools.partial(
    pl.kernel,
    out_type=jax.ShapeDtypeStruct((NP, 8), jnp.float32),
    mesh=_mesh,
    compiler_params=_sc_params,
    scratch_types=[
        pltpu.VMEM((NBATCH, K), jnp.int32),
        pltpu.VMEM((6, K, 8), jnp.float32),
        pltpu.VMEM((ZROWS, 8), jnp.float32),
        pltpu.VMEM_SHARED((NP, 8), jnp.float32),
        pltpu.SemaphoreType.DMA,
    ],
)
def _degree_kernel(ei_hbm, oh_hbm, zeros_hbm, deg_hbm,
                   idx_v, oh_v, zbuf_v, acc_sh, sem):
    cid = lax.axis_index("c")
    sid = lax.axis_index("s")
    wid = cid * NS + sid
    pltpu.sync_copy(oh_hbm, oh_v)
    pltpu.sync_copy(zeros_hbm, zbuf_v)
    for h in range(2):
        pltpu.sync_copy(zbuf_v,
                        acc_sh.at[pl.ds(sid * RPT16 + h * ZROWS, ZROWS)])
    plsc.subcore_barrier()
    for r in range(3):
        for e in range(2):
            pltpu.sync_copy(ei_hbm.at[r, e, sid], idx_v)
            oh = oh_v.at[2 * r + e]

            def body(b):
                pltpu.sync_copy(oh, acc_sh.at[idx_v.at[b]], add=True)
            pl.loop(0, NBATCH)(body)
    plsc.subcore_barrier()
    sl = pl.ds(wid * RPT32, RPT32)
    pltpu.sync_copy(acc_sh.at[sl], deg_hbm.at[sl])


@functools.partial(
    pl.kernel,
    out_type=jax.ShapeDtypeStruct((6, NCHUNK, NP, CW), jnp.float32),
    mesh=_mesh,
    compiler_params=_sc_params,
    scratch_types=[
        pltpu.VMEM((EPT,), jnp.int32),
        pltpu.VMEM((NBATCH, K), jnp.int32),
        pltpu.VMEM((K, CW), jnp.float32),
        pltpu.VMEM((K, CW), jnp.float32),
        pltpu.VMEM((ZROWS, CW), jnp.float32),
        pltpu.VMEM_SHARED((NP, CW), jnp.float32),
        pltpu.SemaphoreType.DMA,
        pltpu.SemaphoreType.DMA,
    ],
)
def _propagate_kernel(y_u_hbm, y_i_hbm, ei_hbm, ei16_hbm, zeros_hbm, agg_hbm,
                      g2_v, sidx_v, rows0_v, rows1_v, zbuf_v, acc_sh,
                      sem0, sem1):
    cid = lax.axis_index("c")
    sid = lax.axis_index("s")
    pltpu.sync_copy(zeros_hbm, zbuf_v)
    f0vec = jnp.full((16,), 0, jnp.int32) + CPS * cid
    ones16 = jnp.full((16,), 1, jnp.int32)
    for p in range(6):
        r = p % 3
        gd, sd = (0, 1) if p < 3 else (1, 0)
        ytab = y_u_hbm if p < 3 else y_i_hbm
        pltpu.sync_copy(ei16_hbm.at[r, gd, sid], g2_v)
        pltpu.sync_copy(ei_hbm.at[r, sd, sid], sidx_v)
        for j in range(CPS):
            f = CPS * cid + j
            inc = f0vec if j == 0 else ones16
            for z in range(2):
                pltpu.sync_copy(
                    zbuf_v, acc_sh.at[pl.ds(sid * RPT16 + z * ZROWS, ZROWS)])

            def tbody(k):
                g2_v[pl.ds(k * 16, 16)] = g2_v[pl.ds(k * 16, 16)] + inc
            pl.loop(0, EPT // 16, unroll=8)(tbody)
            plsc.subcore_barrier()
            ytab_r = ytab.at[r]
            pltpu.async_copy(ytab_r.at[g2_v.at[pl.ds(0, K)]], rows0_v, sem0)

            def body(b):
                pltpu.async_copy(ytab_r.at[g2_v.at[pl.ds((b + 1) * K, K)]],
                                 rows1_v, sem1)
                pltpu.make_async_copy(ytab_r.at[g2_v.at[pl.ds(b * K, K)]],
                                      rows0_v, sem0).wait()
                pltpu.sync_copy(rows0_v, acc_sh.at[sidx_v.at[b]], add=True)

                @pl.when(b + 2 < NBATCH)
                def _():
                    pltpu.async_copy(
                        ytab_r.at[g2_v.at[pl.ds((b + 2) * K, K)]],
                        rows0_v, sem0)
                pltpu.make_async_copy(ytab_r.at[g2_v.at[pl.ds((b + 1) * K, K)]],
                                      rows1_v, sem1).wait()
                pltpu.sync_copy(rows1_v, acc_sh.at[sidx_v.at[b + 1]], add=True)
            pl.loop(0, NBATCH, step=2)(body)
            plsc.subcore_barrier()
            sl = pl.ds(sid * RPT16, RPT16)
            pltpu.sync_copy(acc_sh.at[sl], agg_hbm.at[p, f].at[sl])
            plsc.subcore_barrier()


@functools.partial(
    pl.kernel,
    out_type=[jax.ShapeDtypeStruct((3, B, 128), jnp.float32),
              jax.ShapeDtypeStruct((3, B, 128), jnp.float32),
              jax.ShapeDtypeStruct((B, 8), jnp.float32),
              jax.ShapeDtypeStruct((B, 8), jnp.float32)],
    mesh=_mesh,
    compiler_params=_sc_params,
    scratch_types=[
        pltpu.VMEM((4, K), jnp.int32),
        pltpu.VMEM((4, K), jnp.int32),
        pltpu.VMEM((K, 128), jnp.float32),
        pltpu.VMEM((K, 8), jnp.float32),
        pltpu.SemaphoreType.DMA,
    ],
)
def _gather_kernel(agg_hbm, isdp_u_hbm, isdp_i_hbm, uid_hbm, iid_hbm,
                   gu_hbm, gi_hbm, gisd_u_hbm, gisd_i_hbm,
                   uidx_v, iidx_v, rows_v, rows8_v, sem):
    cid = lax.axis_index("c")
    sid = lax.axis_index("s")
    wid = cid * NS + sid
    pltpu.sync_copy(uid_hbm.at[wid], uidx_v)
    pltpu.sync_copy(iid_hbm.at[wid], iidx_v)
    for b in range(4):
        osl = pl.ds(wid * 4 * K + b * K, K)
        for r in range(3):
            pltpu.async_copy(agg_hbm.at[3 + r].at[uidx_v.at[b]],
                             rows_v, sem).wait()
            pltpu.sync_copy(rows_v, gu_hbm.at[r].at[osl])
            pltpu.async_copy(agg_hbm.at[r].at[iidx_v.at[b]],
                             rows_v, sem).wait()
            pltpu.sync_copy(rows_v, gi_hbm.at[r].at[osl])
        pltpu.async_copy(isdp_u_hbm.at[uidx_v.at[b]], rows8_v, sem).wait()
        pltpu.sync_copy(rows8_v, gisd_u_hbm.at[osl])
        pltpu.async_copy(isdp_i_hbm.at[iidx_v.at[b]], rows8_v, sem).wait()
        pltpu.sync_copy(rows8_v, gisd_i_hbm.at[osl])


def _prep_body(ut_ref, it_ref, deg_ref, yu_ref, yi_ref, pu_ref, pi_ref):
    ut = ut_ref[...]
    it = it_ref[...]
    deg = deg_ref[...]
    isd_u, isd_i = [], []
    for r in range(3):
        isd_u.append(lax.rsqrt(jnp.maximum(deg[:, 2 * r:2 * r + 1], 1.0)))
        isd_i.append(lax.rsqrt(jnp.maximum(deg[:, 2 * r + 1:2 * r + 2], 1.0)))
        yu_ref[r] = ut * isd_u[r]
        yi_ref[r] = it * isd_i[r]
    pad = jnp.ones((isd_u[0].shape[0], 5), jnp.float32)
    pu_ref[...] = jnp.concatenate(isd_u + [pad], axis=1)
    pi_ref[...] = jnp.concatenate(isd_i + [pad], axis=1)


_prep_tc = pl.pallas_call(
    _prep_body,
    grid=(NP // 128,),
    in_specs=[
        pl.BlockSpec((128, 128), lambda i: (i, 0)),
        pl.BlockSpec((128, 128), lambda i: (i, 0)),
        pl.BlockSpec((128, 8), lambda i: (i, 0)),
    ],
    out_specs=[
        pl.BlockSpec((3, 128, 128), lambda i: (0, i, 0)),
        pl.BlockSpec((3, 128, 128), lambda i: (0, i, 0)),
        pl.BlockSpec((128, 8), lambda i: (i, 0)),
        pl.BlockSpec((128, 8), lambda i: (i, 0)),
    ],
    out_shape=[
        jax.ShapeDtypeStruct((3, NP, 128), jnp.float32),
        jax.ShapeDtypeStruct((3, NP, 128), jnp.float32),
        jax.ShapeDtypeStruct((NP, 8), jnp.float32),
        jax.ShapeDtypeStruct((NP, 8), jnp.float32),
    ],
)


def _post1_body(agg_ref, pu_ref, pi_ref, wui_ref, bui_ref, wiu_ref, biu_ref,
                y2u_ref, y2i_ref):
    wui = wui_ref[...]
    bui = bui_ref[...]
    wiu = wiu_ref[...]
    biu = biu_ref[...]
    pu = pu_ref[...]
    pi = pi_ref[...]
    i1 = None
    u1 = None
    for r in range(3):
        ai = agg_ref[r] * pi[:, r:r + 1]
        au = agg_ref[3 + r] * pu[:, r:r + 1]
        oi = jnp.dot(ai, wui[r], preferred_element_type=jnp.float32) \
            + bui[r:r + 1, :]
        ou = jnp.dot(au, wiu[r], preferred_element_type=jnp.float32) \
            + biu[r:r + 1, :]
        i1 = oi if i1 is None else jnp.maximum(i1, oi)
        u1 = ou if u1 is None else jnp.maximum(u1, ou)
    i1 = jnp.maximum(i1, 0.0)
    u1 = jnp.maximum(u1, 0.0)
    for r in range(3):
        y2u_ref[r] = u1 * pu[:, r:r + 1]
        y2i_ref[r] = i1 * pi[:, r:r + 1]


_post1_tc = pl.pallas_call(
    _post1_body,
    grid=(NP // 128,),
    in_specs=[
        pl.BlockSpec((6, 128, 128), lambda i: (0, i, 0)),
        pl.BlockSpec((128, 8), lambda i: (i, 0)),
        pl.BlockSpec((128, 8), lambda i: (i, 0)),
        pl.BlockSpec((3, 128, 128), lambda i: (0, 0, 0)),
        pl.BlockSpec((3, 128), lambda i: (0, 0)),
        pl.BlockSpec((3, 128, 128), lambda i: (0, 0, 0)),
        pl.BlockSpec((3, 128), lambda i: (0, 0)),
    ],
    out_specs=[
        pl.BlockSpec((3, 128, 128), lambda i: (0, i, 0)),
        pl.BlockSpec((3, 128, 128), lambda i: (0, i, 0)),
    ],
    out_shape=[
        jax.ShapeDtypeStruct((3, NP, 128), jnp.float32),
        jax.ShapeDtypeStruct((3, NP, 128), jnp.float32),
    ],
)


def _decoder_body(gu_ref, gi_ref, su_ref, si_ref, wui_ref, bui_ref,
                  wiu_ref, biu_ref, dw1_ref, db1_ref, dw2_ref, db2_ref,
                  out_ref):
    wui = wui_ref[...]
    wiu = wiu_ref[...]
    su = su_ref[...]
    si = si_ref[...]
    u2 = jnp.sum(biu_ref[...], axis=0, keepdims=True)
    i2 = jnp.sum(bui_ref[...], axis=0, keepdims=True)
    for r in range(3):
        au = gu_ref[r] * su[:, r:r + 1]
        ai = gi_ref[r] * si[:, r:r + 1]
        u2 = u2 + jnp.dot(au, wiu[r], preferred_element_type=jnp.float32)
        i2 = i2 + jnp.dot(ai, wui[r], preferred_element_type=jnp.float32)
    dw1 = dw1_ref[...]
    h = (jnp.dot(u2, dw1[:128, :], preferred_element_type=jnp.float32)
         + jnp.dot(i2, dw1[128:, :], preferred_element_type=jnp.float32)
         + db1_ref[...])
    h = jnp.maximum(h, 0.0)
    out_ref[...] = (jnp.dot(h, dw2_ref[...],
                            preferred_element_type=jnp.float32)
                    + db2_ref[...])


_decoder_tc = pl.pallas_call(
    _decoder_body,
    grid=(B // 128,),
    in_specs=[
        pl.BlockSpec((3, 128, 128), lambda i: (0, i, 0)),
        pl.BlockSpec((3, 128, 128), lambda i: (0, i, 0)),
        pl.BlockSpec((128, 8), lambda i: (i, 0)),
        pl.BlockSpec((128, 8), lambda i: (i, 0)),
        pl.BlockSpec((3, 128, 128), lambda i: (0, 0, 0)),
        pl.BlockSpec((3, 128), lambda i: (0, 0)),
        pl.BlockSpec((3, 128, 128), lambda i: (0, 0, 0)),
        pl.BlockSpec((3, 128), lambda i: (0, 0)),
        pl.BlockSpec((256, 128), lambda i: (0, 0)),
        pl.BlockSpec((1, 128), lambda i: (0, 0)),
        pl.BlockSpec((128, 4), lambda i: (0, 0)),
        pl.BlockSpec((1, 4), lambda i: (0, 0)),
    ],
    out_specs=pl.BlockSpec((128, 4), lambda i: (i, 0)),
    out_shape=jax.ShapeDtypeStruct((B, 4), jnp.float32),
)


def kernel(edge_index_view, edge_index_save, edge_index_buy, user_ids,
           item_ids, user_table, item_table,
           W1_ui, b1_ui, W1_iu, b1_iu, W2_ui, b2_ui, W2_iu, b2_iu,
           dec_w1, dec_b1, dec_w2, dec_b2):
    def prep_rel(ei):
        pad = jnp.full((EP - E,), PAD_IDX, jnp.int32)
        s = jnp.concatenate([ei[0].astype(jnp.int32), pad])
        d = jnp.concatenate([ei[1].astype(jnp.int32), pad])
        return jnp.stack([s.reshape(NS, NBATCH, K), d.reshape(NS, NBATCH, K)])

    ei = jnp.stack([prep_rel(edge_index_view), prep_rel(edge_index_save),
                    prep_rel(edge_index_buy)])
    ei16 = (ei * 16).reshape(3, 2, NS, EPT)
    onehots = jnp.broadcast_to(jnp.eye(8, dtype=jnp.float32)[:6, None, :],
                               (6, K, 8))
    zeros_cw = jnp.zeros((ZROWS, CW), jnp.float32)
    ut = jnp.pad(user_table, ((0, NP - NU), (0, 0)))
    it = jnp.pad(item_table, ((0, NP - NI), (0, 0)))
    uid = user_ids.astype(jnp.int32).reshape(NW, 4, K)
    iid = item_ids.astype(jnp.int32).reshape(NW, 4, K)

    deg = _degree_kernel(ei, onehots, zeros_cw)
    y1u, y1i, pu, pi = _prep_tc(ut, it, deg)
    agg1 = _propagate_kernel(y1u.reshape(3, NP * 16, 8),
                             y1i.reshape(3, NP * 16, 8), ei, ei16, zeros_cw)
    agg1_t = jnp.transpose(agg1, (0, 2, 1, 3)).reshape(6, NP, 128)
    y2u, y2i = _post1_tc(agg1_t, pu, pi, W1_ui, b1_ui, W1_iu, b1_iu)
    agg2 = _propagate_kernel(y2u.reshape(3, NP * 16, 8),
                             y2i.reshape(3, NP * 16, 8), ei, ei16, zeros_cw)
    agg2_t = jnp.transpose(agg2, (0, 2, 1, 3)).reshape(6, NP, 128)
    gu, gi, gsu, gsi = _gather_kernel(agg2_t, pu, pi, uid, iid)
    logits = _decoder_tc(gu, gi, gsu, gsi, W2_ui, b2_ui, W2_iu, b2_iu,
                         dec_w1, dec_b1.reshape(1, 128), dec_w2,
                         dec_b2.reshape(1, 4))
    return logits
